# baseline (speedup 1.0000x reference)
import functools

import jax
import jax.numpy as jnp
from jax import lax
from jax.experimental import pallas as pl
from jax.experimental.pallas import tpu as pltpu
from jax.experimental.pallas import tpu_sc as plsc

_N, _D, _E, _K, _F, _NSH, _C = 2048, 768, 64, 2, 1536, 2, 128
_FSH = _F * _NSH
_SLOTS = _E * _C
_SPAD = _SLOTS + _C
_DUMP = _SLOTS
_NW = 32
_TPW = _N // _NW


def _router_body(x_ref, wg_ref, topi_ref, dsta_ref, dstc_ref, wc_ref):
    xb = x_ref[...]
    logits = jnp.dot(xb, wg_ref[...], preferred_element_type=jnp.float32)
    eidx = lax.broadcasted_iota(jnp.int32, (_N, _E), 1)

    m1 = jnp.max(logits, axis=1, keepdims=True)
    i1 = jnp.min(jnp.where(logits == m1, eidx, _E), axis=1, keepdims=True)
    l2 = jnp.where(eidx == i1, -jnp.inf, logits)
    m2 = jnp.max(l2, axis=1, keepdims=True)
    i2 = jnp.min(jnp.where(l2 == m2, eidx, _E), axis=1, keepdims=True)

    w1 = jax.nn.sigmoid(m1 - m2)
    w2 = jax.nn.sigmoid(m2 - m1)

    oh1 = (eidx == i1).astype(jnp.float32)
    oh2 = (eidx == i2).astype(jnp.float32)
    ohc = oh1 + oh2
    r_io = lax.broadcasted_iota(jnp.int32, (_N, _N), 0)
    c_io = lax.broadcasted_iota(jnp.int32, (_N, _N), 1)
    tril = (c_io < r_io).astype(jnp.float32)
    s = jnp.dot(tril, ohc, preferred_element_type=jnp.float32)

    p1 = jnp.sum(s * oh1, axis=1, keepdims=True).astype(jnp.int32)
    p2 = jnp.sum(s * oh2, axis=1, keepdims=True).astype(jnp.int32)

    v1 = p1 < _C
    v2 = p2 < _C
    base1 = i1 * _C
    base2 = i2 * _C
    dsta1 = jnp.where(v1, base1 + p1, _DUMP)
    dsta2 = jnp.where(v2, base2 + p2, _DUMP)
    dstc1 = base1 + jnp.where(v1, p1, 0)
    dstc2 = base2 + jnp.where(v2, p2, 0)

    topi_ref[...] = jnp.concatenate([i1, i2], axis=1)
    dsta_ref[...] = jnp.concatenate([dsta1, dsta2], axis=1)
    dstc_ref[...] = jnp.concatenate([dstc1, dstc2], axis=1)
    wc_ref[...] = jnp.concatenate(
        [w1 * v1.astype(jnp.float32), w2 * v2.astype(jnp.float32)], axis=1)


def _router(x, wg):
    return pl.pallas_call(
        _router_body,
        out_shape=[
            jax.ShapeDtypeStruct((_N, _K), jnp.int32),
            jax.ShapeDtypeStruct((_N, _K), jnp.int32),
            jax.ShapeDtypeStruct((_N, _K), jnp.int32),
            jax.ShapeDtypeStruct((_N, _K), jnp.float32),
        ],
        compiler_params=pltpu.CompilerParams(
            vmem_limit_bytes=100 * 1024 * 1024),
    )(x, wg)


@functools.lru_cache(maxsize=None)
def _sc_mesh():
    return plsc.VectorSubcoreMesh(core_axis_name="c", subcore_axis_name="s")


@functools.lru_cache(maxsize=None)
def _sc_scatter():
    @functools.partial(
        pl.kernel,
        out_type=jax.ShapeDtypeStruct((_SPAD, _D), jnp.float32),
        mesh=_sc_mesh(),
        scratch_types=[
            pltpu.VMEM((_TPW,), jnp.int32),
            pltpu.VMEM((_TPW,), jnp.int32),
            pltpu.VMEM((_TPW, _D), jnp.float32),
            pltpu.SemaphoreType.DMA,
        ],
    )
    def scatter_kernel(x_hbm, d1_hbm, d2_hbm, buf_hbm, i1_v, i2_v, xv, sem):
        wid = lax.axis_index("s") * 2 + lax.axis_index("c")
        pltpu.sync_copy(x_hbm.at[pl.ds(wid * _TPW, _TPW)], xv)
        pltpu.sync_copy(d1_hbm.at[wid], i1_v)
        pltpu.sync_copy(d2_hbm.at[wid], i2_v)
        pltpu.async_copy(xv, buf_hbm.at[i1_v], sem).wait()
        pltpu.async_copy(xv, buf_hbm.at[i2_v], sem).wait()

    return scatter_kernel


def _ffn_body(b_ref, w1_ref, w3_ref, w2_ref, eo_ref):
    xb = b_ref[...]
    g = jnp.dot(xb, w1_ref[...], preferred_element_type=jnp.float32)
    u = jnp.dot(xb, w3_ref[...], preferred_element_type=jnp.float32)
    act = g * jax.nn.sigmoid(g) * u
    eo_ref[...] = jnp.dot(act, w2_ref[...], preferred_element_type=jnp.float32)


def _ffn(buf, w1, w3, w2):
    return pl.pallas_call(
        _ffn_body,
        grid=(_E,),
        in_specs=[
            pl.BlockSpec((_C, _D), lambda e: (e, 0)),
            pl.BlockSpec((None, _D, _F), lambda e: (e, 0, 0)),
            pl.BlockSpec((None, _D, _F), lambda e: (e, 0, 0)),
            pl.BlockSpec((None, _F, _D), lambda e: (e, 0, 0)),
        ],
        out_specs=pl.BlockSpec((_C, _D), lambda e: (e, 0)),
        out_shape=jax.ShapeDtypeStruct((_SLOTS, _D), jnp.float32),
        compiler_params=pltpu.CompilerParams(
            dimension_semantics=("arbitrary",),
            vmem_limit_bytes=100 * 1024 * 1024),
    )(buf, w1, w3, w2)


@functools.lru_cache(maxsize=None)
def _sc_gather():
    @functools.partial(
        pl.kernel,
        out_type=[
            jax.ShapeDtypeStruct((_N, _D), jnp.float32),
            jax.ShapeDtypeStruct((_N, _D), jnp.float32),
        ],
        mesh=_sc_mesh(),
        scratch_types=[
            pltpu.VMEM((_TPW,), jnp.int32),
            pltpu.VMEM((_TPW,), jnp.int32),
            pltpu.VMEM((_TPW, _D), jnp.float32),
            pltpu.VMEM((_TPW, _D), jnp.float32),
            pltpu.SemaphoreType.DMA,
        ],
    )
    def gather_kernel(eo_hbm, c1_hbm, c2_hbm, g1_hbm, g2_hbm, i1_v, i2_v,
                      r1, r2, sem):
        wid = lax.axis_index("s") * 2 + lax.axis_index("c")
        base = wid * _TPW
        pltpu.sync_copy(c1_hbm.at[wid], i1_v)
        pltpu.sync_copy(c2_hbm.at[wid], i2_v)
        pltpu.async_copy(eo_hbm.at[i1_v], r1, sem).wait()
        pltpu.sync_copy(r1, g1_hbm.at[pl.ds(base, _TPW)])
        pltpu.async_copy(eo_hbm.at[i2_v], r2, sem).wait()
        pltpu.sync_copy(r2, g2_hbm.at[pl.ds(base, _TPW)])

    return gather_kernel


_TB = 256


def _shared_body(x_ref, ws1_ref, ws3_ref, ws2_ref, o_ref, b1, b3, b2):
    @pl.when(pl.program_id(0) == 0)
    def _():
        b1[...] = ws1_ref[...].astype(jnp.bfloat16)
        b3[...] = ws3_ref[...].astype(jnp.bfloat16)
        b2[...] = ws2_ref[...].astype(jnp.bfloat16)

    xb = x_ref[...].astype(jnp.bfloat16)
    a = jnp.dot(xb, b1[...], preferred_element_type=jnp.float32)
    b = jnp.dot(xb, b3[...], preferred_element_type=jnp.float32)
    h = a * jax.nn.sigmoid(a) * b
    o_ref[...] = jnp.dot(h.astype(jnp.bfloat16), b2[...],
                         preferred_element_type=jnp.float32)


def _shared(x, ws1, ws3, ws2):
    nblk = _N // _TB
    return pl.pallas_call(
        _shared_body,
        grid=(nblk,),
        in_specs=[
            pl.BlockSpec((_TB, _D), lambda i: (i, 0)),
            pl.BlockSpec((_D, _FSH), lambda i: (0, 0)),
            pl.BlockSpec((_D, _FSH), lambda i: (0, 0)),
            pl.BlockSpec((_FSH, _D), lambda i: (0, 0)),
        ],
        out_specs=pl.BlockSpec((_TB, _D), lambda i: (i, 0)),
        out_shape=jax.ShapeDtypeStruct((_N, _D), jnp.float32),
        scratch_shapes=[
            pltpu.VMEM((_D, _FSH), jnp.bfloat16),
            pltpu.VMEM((_D, _FSH), jnp.bfloat16),
            pltpu.VMEM((_FSH, _D), jnp.bfloat16),
        ],
        compiler_params=pltpu.CompilerParams(
            dimension_semantics=("arbitrary",),
            vmem_limit_bytes=100 * 1024 * 1024),
    )(x, ws1, ws3, ws2)


def _finalize_body(sh_ref, g1_ref, g2_ref, wc_ref, o_ref):
    w1 = wc_ref[:, 0:1]
    w2 = wc_ref[:, 1:2]
    o_ref[...] = sh_ref[...] + g1_ref[...] * w1 + g2_ref[...] * w2


def _finalize(sh, g1, g2, wc):
    nblk = _N // _TB
    return pl.pallas_call(
        _finalize_body,
        grid=(nblk,),
        in_specs=[
            pl.BlockSpec((_TB, _D), lambda i: (i, 0)),
            pl.BlockSpec((_TB, _D), lambda i: (i, 0)),
            pl.BlockSpec((_TB, _D), lambda i: (i, 0)),
            pl.BlockSpec((_TB, _K), lambda i: (i, 0)),
        ],
        out_specs=pl.BlockSpec((_TB, _D), lambda i: (i, 0)),
        out_shape=jax.ShapeDtypeStruct((_N, _D), jnp.float32),
        compiler_params=pltpu.CompilerParams(
            dimension_semantics=("arbitrary",),
            vmem_limit_bytes=100 * 1024 * 1024),
    )(sh, g1, g2, wc)


def kernel(x, Wg, W1, W3, W2, Ws1, Ws3, Ws2):
    topi, dsta, dstc, wc = _router(x, Wg)

    dsta_t = dsta.T.reshape(_K, _NW, _TPW)
    dstc_t = dstc.T.reshape(_K, _NW, _TPW)

    sh = _shared(x, Ws1, Ws3, Ws2)
    buf = _sc_scatter()(x, dsta_t[0], dsta_t[1])
    eo = _ffn(buf, W1, W3, W2)
    g1, g2 = _sc_gather()(eo, dstc_t[0], dstc_t[1])
    out = _finalize(sh, g1, g2, wc)
    return out, topi


# speedup vs baseline: 1.0251x; 1.0251x over previous
import functools

import jax
import jax.numpy as jnp
from jax import lax
from jax.experimental import pallas as pl
from jax.experimental.pallas import tpu as pltpu
from jax.experimental.pallas import tpu_sc as plsc

_N, _D, _E, _K, _F, _NSH, _C = 2048, 768, 64, 2, 1536, 2, 128
_FSH = _F * _NSH
_SLOTS = _E * _C
_SPAD = _SLOTS + _C
_DUMP = _SLOTS
_NW = 32
_TPW = _N // _NW


def _router_body(x_ref, wg_ref, topi_ref, dsta_ref, dstc_ref, wc_ref):
    xb = x_ref[...]
    logits = jnp.dot(xb, wg_ref[...], preferred_element_type=jnp.float32)
    eidx = lax.broadcasted_iota(jnp.int32, (_N, _E), 1)

    m1 = jnp.max(logits, axis=1, keepdims=True)
    i1 = jnp.min(jnp.where(logits == m1, eidx, _E), axis=1, keepdims=True)
    l2 = jnp.where(eidx == i1, -jnp.inf, logits)
    m2 = jnp.max(l2, axis=1, keepdims=True)
    i2 = jnp.min(jnp.where(l2 == m2, eidx, _E), axis=1, keepdims=True)

    w1 = jax.nn.sigmoid(m1 - m2)
    w2 = jax.nn.sigmoid(m2 - m1)

    oh1 = (eidx == i1).astype(jnp.float32)
    oh2 = (eidx == i2).astype(jnp.float32)
    ohc = oh1 + oh2
    r_io = lax.broadcasted_iota(jnp.int32, (_N, _N), 0)
    c_io = lax.broadcasted_iota(jnp.int32, (_N, _N), 1)
    tril = (c_io < r_io).astype(jnp.float32)
    s = jnp.dot(tril, ohc, preferred_element_type=jnp.float32)

    p1 = jnp.sum(s * oh1, axis=1, keepdims=True).astype(jnp.int32)
    p2 = jnp.sum(s * oh2, axis=1, keepdims=True).astype(jnp.int32)

    v1 = p1 < _C
    v2 = p2 < _C
    base1 = i1 * _C
    base2 = i2 * _C
    dsta1 = jnp.where(v1, base1 + p1, _DUMP)
    dsta2 = jnp.where(v2, base2 + p2, _DUMP)
    dstc1 = base1 + jnp.where(v1, p1, 0)
    dstc2 = base2 + jnp.where(v2, p2, 0)

    topi_ref[...] = jnp.concatenate([i1, i2], axis=1)
    dsta_ref[...] = jnp.concatenate([dsta1, dsta2], axis=1)
    dstc_ref[...] = jnp.concatenate([dstc1, dstc2], axis=1)
    wc_ref[...] = jnp.concatenate(
        [w1 * v1.astype(jnp.float32), w2 * v2.astype(jnp.float32)], axis=1)


def _router(x, wg):
    return pl.pallas_call(
        _router_body,
        out_shape=[
            jax.ShapeDtypeStruct((_N, _K), jnp.int32),
            jax.ShapeDtypeStruct((_N, _K), jnp.int32),
            jax.ShapeDtypeStruct((_N, _K), jnp.int32),
            jax.ShapeDtypeStruct((_N, _K), jnp.float32),
        ],
        compiler_params=pltpu.CompilerParams(
            vmem_limit_bytes=100 * 1024 * 1024),
    )(x, wg)


@functools.lru_cache(maxsize=None)
def _sc_mesh():
    return plsc.VectorSubcoreMesh(core_axis_name="c", subcore_axis_name="s")


@functools.lru_cache(maxsize=None)
def _sc_scatter():
    @functools.partial(
        pl.kernel,
        out_type=jax.ShapeDtypeStruct((_SPAD, _D), jnp.float32),
        mesh=_sc_mesh(),
        scratch_types=[
            pltpu.VMEM((_TPW,), jnp.int32),
            pltpu.VMEM((_TPW,), jnp.int32),
            pltpu.VMEM((_TPW, _D), jnp.float32),
            pltpu.SemaphoreType.DMA,
        ],
    )
    def scatter_kernel(x_hbm, d1_hbm, d2_hbm, buf_hbm, i1_v, i2_v, xv, sem):
        wid = lax.axis_index("s") * 2 + lax.axis_index("c")
        pltpu.sync_copy(x_hbm.at[pl.ds(wid * _TPW, _TPW)], xv)
        pltpu.sync_copy(d1_hbm.at[wid], i1_v)
        pltpu.sync_copy(d2_hbm.at[wid], i2_v)
        c1 = pltpu.async_copy(xv, buf_hbm.at[i1_v], sem)
        c2 = pltpu.async_copy(xv, buf_hbm.at[i2_v], sem)
        c1.wait()
        c2.wait()

    return scatter_kernel


def _ffn_body(b_ref, w1_ref, w3_ref, w2_ref, eo_ref):
    xb = b_ref[...]
    g = jnp.dot(xb, w1_ref[...], preferred_element_type=jnp.float32)
    u = jnp.dot(xb, w3_ref[...], preferred_element_type=jnp.float32)
    act = g * jax.nn.sigmoid(g) * u
    eo_ref[...] = jnp.dot(act, w2_ref[...], preferred_element_type=jnp.float32)


def _ffn(buf, w1, w3, w2):
    return pl.pallas_call(
        _ffn_body,
        grid=(_E,),
        in_specs=[
            pl.BlockSpec((_C, _D), lambda e: (e, 0)),
            pl.BlockSpec((None, _D, _F), lambda e: (e, 0, 0)),
            pl.BlockSpec((None, _D, _F), lambda e: (e, 0, 0)),
            pl.BlockSpec((None, _F, _D), lambda e: (e, 0, 0)),
        ],
        out_specs=pl.BlockSpec((_C, _D), lambda e: (e, 0)),
        out_shape=jax.ShapeDtypeStruct((_SLOTS, _D), jnp.float32),
        compiler_params=pltpu.CompilerParams(
            dimension_semantics=("arbitrary",),
            vmem_limit_bytes=100 * 1024 * 1024),
    )(buf, w1, w3, w2)


@functools.lru_cache(maxsize=None)
def _sc_gather():
    @functools.partial(
        pl.kernel,
        out_type=[
            jax.ShapeDtypeStruct((_N, _D), jnp.float32),
            jax.ShapeDtypeStruct((_N, _D), jnp.float32),
        ],
        mesh=_sc_mesh(),
        scratch_types=[
            pltpu.VMEM((_TPW,), jnp.int32),
            pltpu.VMEM((_TPW,), jnp.int32),
            pltpu.VMEM((_TPW, _D), jnp.float32),
            pltpu.VMEM((_TPW, _D), jnp.float32),
            pltpu.SemaphoreType.DMA,
        ],
    )
    def gather_kernel(eo_hbm, c1_hbm, c2_hbm, g1_hbm, g2_hbm, i1_v, i2_v,
                      r1, r2, sem):
        wid = lax.axis_index("s") * 2 + lax.axis_index("c")
        base = wid * _TPW
        pltpu.sync_copy(c1_hbm.at[wid], i1_v)
        pltpu.sync_copy(c2_hbm.at[wid], i2_v)
        c1 = pltpu.async_copy(eo_hbm.at[i1_v], r1, sem)
        c2 = pltpu.async_copy(eo_hbm.at[i2_v], r2, sem)
        c1.wait()
        pltpu.sync_copy(r1, g1_hbm.at[pl.ds(base, _TPW)])
        c2.wait()
        pltpu.sync_copy(r2, g2_hbm.at[pl.ds(base, _TPW)])

    return gather_kernel


_TB = 256


def _combine_body(x_ref, ws1_ref, ws3_ref, ws2_ref, g1_ref, g2_ref, wc_ref,
                  o_ref):
    xb = x_ref[...]
    a = jnp.dot(xb, ws1_ref[...], preferred_element_type=jnp.float32)
    b = jnp.dot(xb, ws3_ref[...], preferred_element_type=jnp.float32)
    h = a * jax.nn.sigmoid(a) * b
    sh = jnp.dot(h, ws2_ref[...], preferred_element_type=jnp.float32)
    w1 = wc_ref[:, 0:1]
    w2 = wc_ref[:, 1:2]
    o_ref[...] = sh + g1_ref[...] * w1 + g2_ref[...] * w2


def _combine(x, ws1, ws3, ws2, g1, g2, wc):
    nblk = _N // _TB
    return pl.pallas_call(
        _combine_body,
        grid=(nblk,),
        in_specs=[
            pl.BlockSpec((_TB, _D), lambda i: (i, 0)),
            pl.BlockSpec((_D, _FSH), lambda i: (0, 0)),
            pl.BlockSpec((_D, _FSH), lambda i: (0, 0)),
            pl.BlockSpec((_FSH, _D), lambda i: (0, 0)),
            pl.BlockSpec((_TB, _D), lambda i: (i, 0)),
            pl.BlockSpec((_TB, _D), lambda i: (i, 0)),
            pl.BlockSpec((_TB, _K), lambda i: (i, 0)),
        ],
        out_specs=pl.BlockSpec((_TB, _D), lambda i: (i, 0)),
        out_shape=jax.ShapeDtypeStruct((_N, _D), jnp.float32),
        compiler_params=pltpu.CompilerParams(
            dimension_semantics=("arbitrary",),
            vmem_limit_bytes=100 * 1024 * 1024),
    )(x, ws1, ws3, ws2, g1, g2, wc)


def kernel(x, Wg, W1, W3, W2, Ws1, Ws3, Ws2):
    topi, dsta, dstc, wc = _router(x, Wg)

    dsta_t = dsta.T.reshape(_K, _NW, _TPW)
    dstc_t = dstc.T.reshape(_K, _NW, _TPW)

    buf = _sc_scatter()(x, dsta_t[0], dsta_t[1])
    eo = _ffn(buf, W1, W3, W2)
    g1, g2 = _sc_gather()(eo, dstc_t[0], dstc_t[1])
    out = _combine(x, Ws1, Ws3, Ws2, g1, g2, wc)
    return out, topi
